# baseline (speedup 1.0000x reference)
import functools

import jax
import jax.numpy as jnp
from jax import lax
from jax.experimental import pallas as pl
from jax.experimental.pallas import tpu as pltpu
from jax.experimental.pallas import tpu_sc as plsc

_FIELDS = 26
_VOCAB = 100000
_EMB = 32
_LANES = 16
_CHUNK = 1024


def _body(n_bchunk, items_per_w, nc, xt_hbm, tab_hbm, out_hbm,
          idx_v, rows_v, sems):
    wid = lax.axis_index("s") * nc + lax.axis_index("c")

    def item_coords(k):
        g = wid * items_per_w + k
        f = g // n_bchunk
        b0 = (g % n_bchunk) * _CHUNK
        return f, b0

    def load_idx(k, slot):
        f, b0 = item_coords(k)
        pltpu.sync_copy(xt_hbm.at[f, pl.ds(b0, _CHUNK)], idx_v.at[slot])
        off = f * _VOCAB

        def add_offs(j, _):
            sl = pl.ds(j * _LANES, _LANES)
            idx_v[slot, sl] = idx_v[slot, sl] + off
            return 0

        lax.fori_loop(0, _CHUNK // _LANES, add_offs, 0, unroll=8)

    def gather(k, slot):
        return pltpu.async_copy(tab_hbm.at[idx_v.at[slot]],
                                rows_v.at[slot], sems.at[slot])

    def writeback(k, slot):
        _, b0 = item_coords(k)
        f, _ = item_coords(k)
        pltpu.sync_copy(rows_v.at[slot], out_hbm.at[pl.ds(b0, _CHUNK), f])

    load_idx(0, 0)
    inflight = {0: gather(0, 0)}
    for k in range(items_per_w):
        nxt = k + 1
        if nxt < items_per_w:
            load_idx(nxt, nxt % 2)
            inflight[nxt] = gather(nxt, nxt % 2)
        inflight.pop(k).wait()
        writeback(k, k % 2)


def kernel(x, tables):
    batch = x.shape[0]
    xt = jnp.swapaxes(x, 0, 1)
    tab_flat = tables.reshape(_FIELDS * _VOCAB, _EMB)

    info = plsc.get_sparse_core_info()
    nc, ns = info.num_cores, info.num_subcores
    nw = nc * ns
    n_bchunk = batch // _CHUNK
    items_per_w = _FIELDS * n_bchunk // nw

    mesh = plsc.VectorSubcoreMesh(core_axis_name="c", subcore_axis_name="s")
    run = pl.kernel(
        functools.partial(_body, n_bchunk, items_per_w, nc),
        out_type=jax.ShapeDtypeStruct((batch, _FIELDS, _EMB), jnp.float32),
        mesh=mesh,
        compiler_params=pltpu.CompilerParams(use_tc_tiling_on_sc=False),
        scratch_types=[
            pltpu.VMEM((2, _CHUNK), jnp.int32),
            pltpu.VMEM((2, _CHUNK, _EMB), jnp.float32),
            pltpu.SemaphoreType.DMA((2,)),
        ],
    )
    out = run(xt, tab_flat)
    return out.reshape(batch, _FIELDS * _EMB)


# speedup vs baseline: 1.0930x; 1.0930x over previous
import functools

import jax
import jax.numpy as jnp
from jax import lax
from jax.experimental import pallas as pl
from jax.experimental.pallas import tpu as pltpu
from jax.experimental.pallas import tpu_sc as plsc

_FIELDS = 26
_VOCAB = 100000
_EMB = 32
_LANES = 16


def _body(rows_per_w, nc, x_hbm, tab_hbm, out_hbm, xv, idx_all, rows_v, sems):
    wid = lax.axis_index("s") * nc + lax.axis_index("c")
    b0 = wid * rows_per_w
    lane = lax.iota(jnp.int32, _LANES)

    pltpu.sync_copy(x_hbm.at[pl.ds(b0, rows_per_w)], xv)

    for f in range(_FIELDS):
        col = jnp.full((_LANES,), f, jnp.int32)
        off = f * _VOCAB

        def extract(j, _):
            rows = j * _LANES + lane
            vals = plsc.load_gather(xv, [rows, col])
            idx_all[f, pl.ds(j * _LANES, _LANES)] = vals + off
            return 0

        lax.fori_loop(0, rows_per_w // _LANES, extract, 0, unroll=4)

    def gather(f, slot):
        return pltpu.async_copy(tab_hbm.at[idx_all.at[f]],
                                rows_v.at[slot], sems.at[slot])

    def writeback(f, slot):
        pltpu.sync_copy(rows_v.at[slot],
                        out_hbm.at[pl.ds(b0, rows_per_w),
                                   pl.ds(f * _EMB, _EMB)])

    inflight = {0: gather(0, 0)}
    for f in range(_FIELDS):
        nxt = f + 1
        if nxt < _FIELDS:
            inflight[nxt] = gather(nxt, nxt % 2)
        inflight.pop(f).wait()
        writeback(f, f % 2)


def kernel(x, tables):
    batch = x.shape[0]
    tab_flat = tables.reshape(_FIELDS * _VOCAB, _EMB)

    info = plsc.get_sparse_core_info()
    nc, ns = info.num_cores, info.num_subcores
    nw = nc * ns
    rows_per_w = batch // nw

    mesh = plsc.VectorSubcoreMesh(core_axis_name="c", subcore_axis_name="s")
    run = pl.kernel(
        functools.partial(_body, rows_per_w, nc),
        out_type=jax.ShapeDtypeStruct((batch, _FIELDS * _EMB), jnp.float32),
        mesh=mesh,
        compiler_params=pltpu.CompilerParams(use_tc_tiling_on_sc=False,
                                             needs_layout_passes=False),
        scratch_types=[
            pltpu.VMEM((512, _FIELDS), jnp.int32),
            pltpu.VMEM((_FIELDS, 512), jnp.int32),
            pltpu.VMEM((2, 512, _EMB), jnp.float32),
            pltpu.SemaphoreType.DMA((2,)),
        ],
    )
    return run(x, tab_flat)
